# baseline (speedup 1.0000x reference)
import functools

import jax
import jax.numpy as jnp
from jax import lax
from jax.experimental import pallas as pl
from jax.experimental.pallas import tpu as pltpu
from jax.experimental.pallas import tpu_sc as plsc

NUM_CAT = 26
N_FIELDS = 39
N_NUM = N_FIELDS - NUM_CAT
DIM = 128
BATCH = 16384

NC, NS = 2, 16
NW = NC * NS
BT = BATCH * NUM_CAT
OUT_ROWS = BATCH * N_FIELDS
PW = BT // NW
CH = 128
NCH = PW // CH

_sc_mesh = plsc.VectorSubcoreMesh(core_axis_name="c", subcore_axis_name="s")


@functools.partial(
    pl.kernel,
    out_type=jax.ShapeDtypeStruct((OUT_ROWS, DIM), jnp.float32),
    mesh=_sc_mesh,
    scratch_types=[
        pltpu.VMEM((NCH, CH), jnp.int32),
        pltpu.VMEM((NCH, CH), jnp.int32),
        pltpu.VMEM((CH, DIM), jnp.float32),
        pltpu.SemaphoreType.DMA,
        pltpu.SemaphoreType.DMA,
    ],
)
def _sc_gather_place(table_hbm, idx_hbm, dst_hbm, out_hbm,
                     idx_v, dst_v, rows_v, gsem, wsem):
    wid = lax.axis_index("s") * NC + lax.axis_index("c")
    pltpu.sync_copy(idx_hbm.at[wid], idx_v)
    pltpu.sync_copy(dst_hbm.at[wid], dst_v)

    def chunk(c, carry):
        pltpu.async_copy(table_hbm.at[idx_v.at[c]], rows_v, gsem).wait()
        pltpu.async_copy(rows_v, out_hbm.at[dst_v.at[c]], wsem).wait()
        return carry

    lax.fori_loop(0, NCH, chunk, 0)


def _numfill_body(src_ref, num_ref, w_ref, b_ref, out_ref):
    del src_ref
    w = w_ref[0, :]
    b = b_ref[0, :]
    dense = jnp.maximum(
        num_ref[...][:, :, None] * w[None, None, :] + b[None, None, :], 0.0
    )
    out_ref[...] = dense[:, None, :, :]


def _numfill(buf4, num, w, b):
    BB = 512
    grid = BATCH // BB
    return pl.pallas_call(
        _numfill_body,
        grid=(grid,),
        in_specs=[
            pl.BlockSpec(memory_space=pl.ANY),
            pl.BlockSpec((BB, N_NUM), lambda i: (i, 0)),
            pl.BlockSpec((1, DIM), lambda i: (0, 0)),
            pl.BlockSpec((1, DIM), lambda i: (0, 0)),
        ],
        out_specs=pl.BlockSpec((BB, 1, N_NUM, DIM), lambda i: (i, 2, 0, 0)),
        out_shape=jax.ShapeDtypeStruct((BATCH, 3, N_NUM, DIM), jnp.float32),
        input_output_aliases={0: 0},
    )(buf4, num, w, b)


def kernel(x, table, W_num, b_num):
    idx = x[:, :NUM_CAT].astype(jnp.int32).reshape(NW, NCH, CH)
    num = x[:, NUM_CAT:]
    p = jnp.arange(BT, dtype=jnp.int32)
    dst = ((p // NUM_CAT) * N_FIELDS + p % NUM_CAT).reshape(NW, NCH, CH)
    buf = _sc_gather_place(table, idx, dst)
    buf4 = buf.reshape(BATCH, 3, N_NUM, DIM)
    out = _numfill(buf4, num, W_num.reshape(1, DIM), b_num.reshape(1, DIM))
    return out.reshape(BATCH, N_FIELDS, DIM)


# speedup vs baseline: 1.2186x; 1.2186x over previous
import functools

import jax
import jax.numpy as jnp
from jax import lax
from jax.experimental import pallas as pl
from jax.experimental.pallas import tpu as pltpu
from jax.experimental.pallas import tpu_sc as plsc

NUM_CAT = 26
N_FIELDS = 39
N_STRIDE = 40
N_NUM = N_FIELDS - NUM_CAT
DIM = 128
BATCH = 16384

NC, NS = 2, 16
NW = NC * NS
BT = BATCH * NUM_CAT
STAGE_ROWS = BATCH * N_STRIDE
PW = BT // NW
CH = 128
NCH = PW // CH
NBUF = 4
NITER = NCH // NBUF

_sc_mesh = plsc.VectorSubcoreMesh(core_axis_name="c", subcore_axis_name="s")


@functools.partial(
    pl.kernel,
    out_type=jax.ShapeDtypeStruct((STAGE_ROWS, DIM), jnp.float32),
    mesh=_sc_mesh,
    scratch_types=[
        pltpu.VMEM((NCH, CH), jnp.int32),
        pltpu.VMEM((NCH, CH), jnp.int32),
    ]
    + [pltpu.VMEM((CH, DIM), jnp.float32) for _ in range(NBUF)]
    + [pltpu.SemaphoreType.DMA for _ in range(2 * NBUF)],
)
def _sc_gather_place(table_hbm, idx_hbm, dst_hbm, out_hbm, idx_v, dst_v, *bufs_sems):
    rows = bufs_sems[:NBUF]
    gsem = bufs_sems[NBUF : 2 * NBUF]
    wsem = bufs_sems[2 * NBUF :]
    wid = lax.axis_index("s") * NC + lax.axis_index("c")
    pltpu.sync_copy(idx_hbm.at[wid], idx_v)
    pltpu.sync_copy(dst_hbm.at[wid], dst_v)

    def body(i, carry):
        for j in range(NBUF):
            c = i * NBUF + j

            @pl.when(i > 0)
            def _():
                pltpu.make_async_copy(
                    rows[j], out_hbm.at[dst_v.at[c - NBUF]], wsem[j]
                ).wait()

            pltpu.async_copy(table_hbm.at[idx_v.at[c]], rows[j], gsem[j])
        for j in range(NBUF):
            c = i * NBUF + j
            pltpu.make_async_copy(table_hbm.at[idx_v.at[c]], rows[j], gsem[j]).wait()
            pltpu.async_copy(rows[j], out_hbm.at[dst_v.at[c]], wsem[j])
        return carry

    lax.fori_loop(0, NITER, body, 0)
    for j in range(NBUF):
        c = NCH - NBUF + j
        pltpu.make_async_copy(rows[j], out_hbm.at[dst_v.at[c]], wsem[j]).wait()


def _combine_body(stage_ref, num_ref, w_ref, b_ref, out_ref):
    BB = out_ref.shape[0]
    cat = stage_ref[...].reshape(BB, N_STRIDE, DIM)[:, :NUM_CAT, :]
    w = w_ref[0, :]
    b = b_ref[0, :]
    dense = jnp.maximum(
        num_ref[...][:, :, None] * w[None, None, :] + b[None, None, :], 0.0
    )
    out_ref[...] = jnp.concatenate([cat, dense], axis=1)


def _combine(stage, num, w, b):
    BB = 256
    grid = BATCH // BB
    return pl.pallas_call(
        _combine_body,
        grid=(grid,),
        in_specs=[
            pl.BlockSpec((BB * N_STRIDE, DIM), lambda i: (i, 0)),
            pl.BlockSpec((BB, N_NUM), lambda i: (i, 0)),
            pl.BlockSpec((1, DIM), lambda i: (0, 0)),
            pl.BlockSpec((1, DIM), lambda i: (0, 0)),
        ],
        out_specs=pl.BlockSpec((BB, N_FIELDS, DIM), lambda i: (i, 0, 0)),
        out_shape=jax.ShapeDtypeStruct((BATCH, N_FIELDS, DIM), jnp.float32),
    )(stage, num, w, b)


def kernel(x, table, W_num, b_num):
    idx = x[:, :NUM_CAT].astype(jnp.int32).reshape(NW, NCH, CH)
    num = x[:, NUM_CAT:]
    p = jnp.arange(BT, dtype=jnp.int32)
    dst = ((p // NUM_CAT) * N_STRIDE + p % NUM_CAT).reshape(NW, NCH, CH)
    stage = _sc_gather_place(table, idx, dst)
    return _combine(stage, num, W_num.reshape(1, DIM), b_num.reshape(1, DIM))


# speedup vs baseline: 2.1579x; 1.7708x over previous
import functools

import jax
import jax.numpy as jnp
from jax import lax
from jax.experimental import pallas as pl
from jax.experimental.pallas import tpu as pltpu
from jax.experimental.pallas import tpu_sc as plsc

NUM_CAT = 26
N_FIELDS = 39
N_NUM = N_FIELDS - NUM_CAT
DIM = 128
BATCH = 16384

NC, NS = 2, 16
NW = NC * NS
BT = BATCH * NUM_CAT
STAGE_ROWS = BATCH * N_FIELDS
PW = BT // NW
CH = 128
NCH = PW // CH
NBUF = 4
NITER = NCH // NBUF

_sc_mesh = plsc.VectorSubcoreMesh(core_axis_name="c", subcore_axis_name="s")


@functools.partial(
    pl.kernel,
    out_type=jax.ShapeDtypeStruct((STAGE_ROWS, DIM), jnp.float32),
    mesh=_sc_mesh,
    scratch_types=[
        pltpu.VMEM((NCH, CH), jnp.int32),
    ]
    + [pltpu.VMEM((CH, DIM), jnp.float32) for _ in range(NBUF)]
    + [pltpu.SemaphoreType.DMA for _ in range(2 * NBUF)],
)
def _sc_gather(table_hbm, idx_hbm, out_hbm, idx_v, *bufs_sems):
    rows = bufs_sems[:NBUF]
    gsem = bufs_sems[NBUF : 2 * NBUF]
    wsem = bufs_sems[2 * NBUF :]
    wid = lax.axis_index("s") * NC + lax.axis_index("c")
    base = wid * PW
    pltpu.sync_copy(idx_hbm.at[wid], idx_v)

    def body(i, carry):
        for j in range(NBUF):
            c = i * NBUF + j

            @pl.when(i > 0)
            def _():
                pltpu.make_async_copy(
                    rows[j], out_hbm.at[pl.ds(base + (c - NBUF) * CH, CH)], wsem[j]
                ).wait()

            pltpu.async_copy(table_hbm.at[idx_v.at[c]], rows[j], gsem[j])
        for j in range(NBUF):
            c = i * NBUF + j
            pltpu.make_async_copy(table_hbm.at[idx_v.at[c]], rows[j], gsem[j]).wait()
            pltpu.async_copy(rows[j], out_hbm.at[pl.ds(base + c * CH, CH)], wsem[j])
        return carry

    lax.fori_loop(0, NITER, body, 0)
    for j in range(NBUF):
        c = NCH - NBUF + j
        pltpu.make_async_copy(
            rows[j], out_hbm.at[pl.ds(base + c * CH, CH)], wsem[j]
        ).wait()


_NB = BATCH // 2


def _numfill_body(src_ref, num_ref, w_ref, b_ref, out_ref):
    del src_ref
    i = pl.program_id(0)
    field = i // 2
    half = i % 2
    numv = num_ref[field, pl.ds(half * _NB, _NB)]
    w = w_ref[0, :]
    b = b_ref[0, :]
    out_ref[...] = jnp.maximum(numv[:, None] * w[None, :] + b[None, :], 0.0)


def _numfill(stage, numT, w, b):
    grid = 2 * N_NUM
    return pl.pallas_call(
        _numfill_body,
        grid=(grid,),
        in_specs=[
            pl.BlockSpec(memory_space=pl.ANY),
            pl.BlockSpec((N_NUM, BATCH), lambda i: (0, 0)),
            pl.BlockSpec((1, DIM), lambda i: (0, 0)),
            pl.BlockSpec((1, DIM), lambda i: (0, 0)),
        ],
        out_specs=pl.BlockSpec((_NB, DIM), lambda i: (i + 2 * NUM_CAT, 0)),
        out_shape=jax.ShapeDtypeStruct((STAGE_ROWS, DIM), jnp.float32),
        input_output_aliases={0: 0},
    )(stage, numT, w, b)


def kernel(x, table, W_num, b_num):
    idxT = x[:, :NUM_CAT].astype(jnp.int32).T.reshape(NW, NCH, CH)
    numT = x[:, NUM_CAT:].T
    stage = _sc_gather(table, idxT)
    full = _numfill(stage, numT, W_num.reshape(1, DIM), b_num.reshape(1, DIM))
    out3 = full.reshape(N_FIELDS, BATCH, DIM)
    return jnp.transpose(out3, (1, 0, 2))


# speedup vs baseline: 7.6810x; 3.5594x over previous
import functools

import jax
import jax.numpy as jnp
from jax import lax
from jax.experimental import pallas as pl
from jax.experimental.pallas import tpu as pltpu
from jax.experimental.pallas import tpu_sc as plsc

NUM_CAT = 26
N_FIELDS = 39
N_NUM = N_FIELDS - NUM_CAT
DIM = 128
BATCH = 16384

NC, NS = 2, 16
NW = NC * NS
BT = BATCH * NUM_CAT
STAGE_ROWS = BATCH * N_FIELDS
PW = BT // NW
CH = 128
NCH = PW // CH
NBUF = 4
NITER = NCH // NBUF

_sc_mesh = plsc.VectorSubcoreMesh(core_axis_name="c", subcore_axis_name="s")


@functools.partial(
    pl.kernel,
    out_type=jax.ShapeDtypeStruct((STAGE_ROWS, DIM), jnp.float32),
    mesh=_sc_mesh,
    scratch_types=[
        pltpu.VMEM_SHARED((256, DIM), jnp.float32),
        pltpu.VMEM((NCH, CH), jnp.int32),
    ]
    + [pltpu.VMEM((CH, DIM), jnp.float32) for _ in range(NBUF)]
    + [pltpu.SemaphoreType.DMA for _ in range(2 * NBUF)],
)
def _sc_gather(table_hbm, idx_hbm, out_hbm, tab_sh, idx_v, *bufs_sems):
    rows = bufs_sems[:NBUF]
    gsem = bufs_sems[NBUF : 2 * NBUF]
    wsem = bufs_sems[2 * NBUF :]
    wid = lax.axis_index("s") * NC + lax.axis_index("c")
    base = wid * PW
    pltpu.sync_copy(idx_hbm.at[wid], idx_v)

    @pl.when(lax.axis_index("s") == 0)
    def _():
        pltpu.sync_copy(table_hbm, tab_sh)

    plsc.subcore_barrier()

    def body(i, carry):
        for j in range(NBUF):
            c = i * NBUF + j

            @pl.when(i > 0)
            def _():
                pltpu.make_async_copy(
                    rows[j], out_hbm.at[pl.ds(base + (c - NBUF) * CH, CH)], wsem[j]
                ).wait()

            pltpu.async_copy(tab_sh.at[idx_v.at[c]], rows[j], gsem[j])
        for j in range(NBUF):
            c = i * NBUF + j
            pltpu.make_async_copy(tab_sh.at[idx_v.at[c]], rows[j], gsem[j]).wait()
            pltpu.async_copy(rows[j], out_hbm.at[pl.ds(base + c * CH, CH)], wsem[j])
        return carry

    lax.fori_loop(0, NITER, body, 0)
    for j in range(NBUF):
        c = NCH - NBUF + j
        pltpu.make_async_copy(
            rows[j], out_hbm.at[pl.ds(base + c * CH, CH)], wsem[j]
        ).wait()


_NB = BATCH // 2


def _numfill_body(src_ref, num_ref, w_ref, b_ref, out_ref):
    del src_ref
    i = pl.program_id(0)
    field = i // 2
    half = i % 2
    numv = num_ref[field, pl.ds(half * _NB, _NB)]
    w = w_ref[0, :]
    b = b_ref[0, :]
    out_ref[...] = jnp.maximum(numv[:, None] * w[None, :] + b[None, :], 0.0)


def _numfill(stage, numT, w, b):
    grid = 2 * N_NUM
    return pl.pallas_call(
        _numfill_body,
        grid=(grid,),
        in_specs=[
            pl.BlockSpec(memory_space=pl.ANY),
            pl.BlockSpec((N_NUM, BATCH), lambda i: (0, 0)),
            pl.BlockSpec((1, DIM), lambda i: (0, 0)),
            pl.BlockSpec((1, DIM), lambda i: (0, 0)),
        ],
        out_specs=pl.BlockSpec((_NB, DIM), lambda i: (i + 2 * NUM_CAT, 0)),
        out_shape=jax.ShapeDtypeStruct((STAGE_ROWS, DIM), jnp.float32),
        input_output_aliases={0: 0},
    )(stage, numT, w, b)


def kernel(x, table, W_num, b_num):
    idxT = x[:, :NUM_CAT].astype(jnp.int32).T.reshape(NW, NCH, CH)
    numT = x[:, NUM_CAT:].T
    stage = _sc_gather(table, idxT)
    full = _numfill(stage, numT, W_num.reshape(1, DIM), b_num.reshape(1, DIM))
    out3 = full.reshape(N_FIELDS, BATCH, DIM)
    return jnp.transpose(out3, (1, 0, 2))


# speedup vs baseline: 7.6818x; 1.0001x over previous
import functools

import jax
import jax.numpy as jnp
from jax import lax
from jax.experimental import pallas as pl
from jax.experimental.pallas import tpu as pltpu
from jax.experimental.pallas import tpu_sc as plsc

NUM_CAT = 26
N_FIELDS = 39
N_NUM = N_FIELDS - NUM_CAT
DIM = 128
BATCH = 16384

NC, NS = 2, 16
NW = NC * NS
BT = BATCH * NUM_CAT
STAGE_ROWS = BATCH * N_FIELDS
PW = BT // NW
CH = 128
NCH = PW // CH
NBUF = 4
NITER = NCH // NBUF

_sc_mesh = plsc.VectorSubcoreMesh(core_axis_name="c", subcore_axis_name="s")


@functools.partial(
    pl.kernel,
    out_type=jax.ShapeDtypeStruct((STAGE_ROWS, DIM), jnp.float32),
    mesh=_sc_mesh,
    scratch_types=[
        pltpu.VMEM_SHARED((256, DIM), jnp.float32),
        pltpu.VMEM((NCH, CH), jnp.int32),
    ]
    + [pltpu.VMEM((CH, DIM), jnp.float32) for _ in range(NBUF)]
    + [pltpu.SemaphoreType.DMA for _ in range(2 * NBUF)],
)
def _sc_gather(table_hbm, idx_hbm, out_hbm, tab_sh, idx_v, *bufs_sems):
    rows = bufs_sems[:NBUF]
    gsem = bufs_sems[NBUF : 2 * NBUF]
    wsem = bufs_sems[2 * NBUF :]
    wid = lax.axis_index("s") * NC + lax.axis_index("c")
    base = wid * PW
    pltpu.sync_copy(idx_hbm.at[wid], idx_v)

    @pl.when(lax.axis_index("s") == 0)
    def _():
        pltpu.sync_copy(table_hbm, tab_sh)

    plsc.subcore_barrier()

    def body(i, carry):
        for j in range(NBUF):
            c = i * NBUF + j

            @pl.when(i > 0)
            def _():
                pltpu.make_async_copy(
                    rows[j], out_hbm.at[pl.ds(base + (c - NBUF) * CH, CH)], wsem[j]
                ).wait()

            pltpu.async_copy(tab_sh.at[idx_v.at[c]], rows[j], gsem[j])
        for j in range(NBUF):
            c = i * NBUF + j
            pltpu.make_async_copy(tab_sh.at[idx_v.at[c]], rows[j], gsem[j]).wait()
            pltpu.async_copy(rows[j], out_hbm.at[pl.ds(base + c * CH, CH)], wsem[j])
        return carry

    lax.fori_loop(0, NITER, body, 0)
    for j in range(NBUF):
        c = NCH - NBUF + j
        pltpu.make_async_copy(
            rows[j], out_hbm.at[pl.ds(base + c * CH, CH)], wsem[j]
        ).wait()


_NB = BATCH // 2


def _numfill_body(src_ref, num_ref, w_ref, b_ref, out_ref):
    del src_ref
    i = pl.program_id(0)
    field = i // 2
    half = i % 2
    numv = num_ref[field, pl.ds(half * _NB, _NB)]
    w = w_ref[0, :]
    b = b_ref[0, :]
    out_ref[...] = jnp.maximum(numv[:, None] * w[None, :] + b[None, :], 0.0)


def _numfill(stage, numT, w, b):
    grid = 2 * N_NUM
    return pl.pallas_call(
        _numfill_body,
        grid=(grid,),
        in_specs=[
            pl.BlockSpec(memory_space=pl.ANY),
            pl.BlockSpec((N_NUM, BATCH), lambda i: (0, 0)),
            pl.BlockSpec((1, DIM), lambda i: (0, 0)),
            pl.BlockSpec((1, DIM), lambda i: (0, 0)),
        ],
        out_specs=pl.BlockSpec((_NB, DIM), lambda i: (i + 2 * NUM_CAT, 0)),
        out_shape=jax.ShapeDtypeStruct((STAGE_ROWS, DIM), jnp.float32),
        input_output_aliases={0: 0},
    )(stage, numT, w, b)


def kernel(x, table, W_num, b_num):
    xT = x.T
    idxT = xT[:NUM_CAT].astype(jnp.int32).reshape(NW, NCH, CH)
    numT = xT[NUM_CAT:]
    stage = _sc_gather(table, idxT)
    full = _numfill(stage, numT, W_num.reshape(1, DIM), b_num.reshape(1, DIM))
    out3 = full.reshape(N_FIELDS, BATCH, DIM)
    return jnp.transpose(out3, (1, 0, 2))
